# baseline (speedup 1.0000x reference)
import functools

import jax
import jax.numpy as jnp
from jax import lax
from jax.experimental import pallas as pl
from jax.experimental.pallas import tpu as pltpu
from jax.experimental.pallas import tpu_sc as plsc

NC, NS = 2, 16
NW = NC * NS
LANES = 16


def _make_lookup(batch, seq_len, latent_dim):
    ch = seq_len // NW
    nsl = latent_dim // LANES
    mesh = plsc.VectorSubcoreMesh(core_axis_name="c", subcore_axis_name="s")

    @functools.partial(
        pl.kernel,
        out_type=jax.ShapeDtypeStruct((batch * seq_len, latent_dim), jnp.float32),
        mesh=mesh,
        scratch_types=[
            pltpu.VMEM((batch, ch), jnp.int32),
            pltpu.VMEM((ch, latent_dim), jnp.float32),
            pltpu.VMEM((2, ch // 2, latent_dim), jnp.float32),
            [pltpu.SemaphoreType.DMA] * 2,
            [pltpu.SemaphoreType.DMA] * 2,
        ],
    )
    def body(tok_hbm, emb_hbm, pos_hbm, out_hbm, idx_v, pbuf, bufs, gsem, ssem):
        wid = lax.axis_index("s") * NC + lax.axis_index("c")
        s0 = wid * ch
        sub = ch // 2
        nt = batch * 2
        pltpu.sync_copy(pos_hbm.at[pl.ds(s0, ch)], pbuf)
        for b in range(batch):
            pltpu.sync_copy(tok_hbm.at[b * NW + wid], idx_v.at[b])

        def gather(t, k):
            b, h = t // 2, t % 2
            return pltpu.async_copy(
                emb_hbm.at[idx_v.at[b, pl.ds(h * sub, sub)]], bufs.at[k], gsem[k]
            )

        gathers = [None] * nt
        stores = [None] * nt
        gathers[0] = gather(0, 0)
        for t in range(nt):
            k = t % 2
            b, h = t // 2, t % 2
            buf = bufs.at[k]
            gathers[t].wait()
            if t + 1 < nt:
                if t >= 1:
                    stores[t - 1].wait()
                gathers[t + 1] = gather(t + 1, 1 - k)

            def row(r2, _, buf=buf, h=h):
                for u in range(2):
                    r = r2 * 2 + u
                    for j in range(nsl):
                        sl = pl.ds(j * LANES, LANES)
                        buf[r, sl] = buf[r, sl] + pbuf[h * sub + r, sl]
                return 0

            lax.fori_loop(0, sub // 2, row, 0)
            stores[t] = pltpu.async_copy(
                buf, out_hbm.at[pl.ds(b * seq_len + s0 + h * sub, sub)], ssem[k]
            )
        stores[nt - 2].wait()
        stores[nt - 1].wait()

    return body


def kernel(tokens, embed_table, pos_table):
    b, s = tokens.shape
    v, d = embed_table.shape
    ch = s // NW
    tok = tokens.reshape(b * NW, ch).astype(jnp.int32)
    out = _make_lookup(b, s, d)(tok, embed_table, pos_table)
    return out.reshape(b, s, d)


# speedup vs baseline: 1.1783x; 1.1783x over previous
import functools

import jax
import jax.numpy as jnp
from jax import lax
from jax.experimental import pallas as pl
from jax.experimental.pallas import tpu as pltpu
from jax.experimental.pallas import tpu_sc as plsc

NC, NS = 2, 16
NW = NC * NS
LANES = 16


def _make_lookup(batch, seq_len, latent_dim):
    ch = seq_len // NW
    nsl = latent_dim // LANES
    mesh = plsc.VectorSubcoreMesh(core_axis_name="c", subcore_axis_name="s")

    @functools.partial(
        pl.kernel,
        out_type=jax.ShapeDtypeStruct((batch * seq_len, latent_dim), jnp.float32),
        mesh=mesh,
        scratch_types=[
            pltpu.VMEM((batch, ch), jnp.int32),
            pltpu.VMEM((ch, latent_dim), jnp.float32),
            pltpu.VMEM((ch, latent_dim), jnp.float32),
            pltpu.SemaphoreType.DMA,
        ],
    )
    def body(tok_hbm, emb_hbm, pos_hbm, out_hbm, idx_v, pbuf, buf, sem):
        wid = lax.axis_index("s") * NC + lax.axis_index("c")
        s0 = wid * ch
        pltpu.sync_copy(pos_hbm.at[pl.ds(s0, ch)], pbuf)
        for b in range(batch):
            pltpu.sync_copy(tok_hbm.at[b, pl.ds(s0, ch)], idx_v.at[b])
        for b in range(batch):
            pltpu.async_copy(emb_hbm.at[idx_v.at[b]], buf, sem).wait()

            def row(r2, _):
                for u in range(2):
                    r = r2 * 2 + u
                    for j in range(nsl):
                        sl = pl.ds(j * LANES, LANES)
                        buf[r, sl] = buf[r, sl] + pbuf[r, sl]
                return 0

            lax.fori_loop(0, ch // 2, row, 0)
            pltpu.sync_copy(buf, out_hbm.at[pl.ds(b * seq_len + s0, ch)])

    return body


def kernel(tokens, embed_table, pos_table):
    b, s = tokens.shape
    v, d = embed_table.shape
    out = _make_lookup(b, s, d)(tokens.astype(jnp.int32), embed_table, pos_table)
    return out.reshape(b, s, d)


# speedup vs baseline: 1.3286x; 1.1276x over previous
import functools

import jax
import jax.numpy as jnp
from jax import lax
from jax.experimental import pallas as pl
from jax.experimental.pallas import tpu as pltpu
from jax.experimental.pallas import tpu_sc as plsc

NC, NS = 2, 16
NW = NC * NS
LANES = 16
GP = 16


def _make_lookup(batch, seq_len, latent_dim):
    ch = seq_len // NW
    ng = ch // GP
    nsl = latent_dim // LANES
    half = nsl // 2
    mesh = plsc.VectorSubcoreMesh(core_axis_name="c", subcore_axis_name="s")

    @functools.partial(
        pl.kernel,
        out_type=jax.ShapeDtypeStruct((batch * seq_len, latent_dim), jnp.float32),
        mesh=mesh,
        scratch_types=[
            pltpu.VMEM((batch, ch), jnp.int32),
            pltpu.VMEM((batch * GP,), jnp.int32),
            pltpu.VMEM((GP, latent_dim), jnp.float32),
            pltpu.VMEM((batch * GP, latent_dim), jnp.float32),
            pltpu.SemaphoreType.DMA,
            pltpu.SemaphoreType.DMA,
        ],
    )
    def body(tok_hbm, emb_hbm, pos_hbm, out_hbm, idx_v, gidx, pbuf, buf, gsem, ssem):
        wid = lax.axis_index("s") * NC + lax.axis_index("c")
        s0 = wid * ch
        for b in range(batch):
            pltpu.sync_copy(tok_hbm.at[b * NW + wid], idx_v.at[b])
        for g in range(ng):
            for b in range(batch):
                gidx[pl.ds(b * GP, GP)] = idx_v[b, pl.ds(g * GP, GP)]
            gather = pltpu.async_copy(emb_hbm.at[gidx], buf, gsem)
            pltpu.sync_copy(pos_hbm.at[pl.ds(s0 + g * GP, GP)], pbuf)
            gather.wait()

            def pos_row(p, _):
                for h in range(2):
                    prow = [
                        pbuf[p, pl.ds((h * half + j) * LANES, LANES)]
                        for j in range(half)
                    ]
                    for b in range(batch):
                        r = b * GP + p
                        for j in range(half):
                            sl = pl.ds((h * half + j) * LANES, LANES)
                            buf[r, sl] = buf[r, sl] + prow[j]
                return 0

            lax.fori_loop(0, GP, pos_row, 0)
            stores = [
                pltpu.async_copy(
                    buf.at[pl.ds(b * GP, GP)],
                    out_hbm.at[pl.ds(b * seq_len + s0 + g * GP, GP)],
                    ssem,
                )
                for b in range(batch)
            ]
            for st in stores:
                st.wait()

    return body


def kernel(tokens, embed_table, pos_table):
    b, s = tokens.shape
    v, d = embed_table.shape
    tok = tokens.reshape(b * NW, s // NW).astype(jnp.int32)
    out = _make_lookup(b, s, d)(tok, embed_table, pos_table)
    return out.reshape(b, s, d)


# speedup vs baseline: 1.5654x; 1.1783x over previous
import functools

import jax
import jax.numpy as jnp
from jax import lax
from jax.experimental import pallas as pl
from jax.experimental.pallas import tpu as pltpu
from jax.experimental.pallas import tpu_sc as plsc

NC, NS = 2, 16
NW = NC * NS
LANES = 16
GP = 16


def _make_lookup(batch, seq_len, latent_dim):
    ch = seq_len // NW
    ng = ch // GP
    nsl = latent_dim // LANES
    half = nsl // 2
    mesh = plsc.VectorSubcoreMesh(core_axis_name="c", subcore_axis_name="s")

    @functools.partial(
        pl.kernel,
        out_type=jax.ShapeDtypeStruct((batch * seq_len, latent_dim), jnp.float32),
        mesh=mesh,
        scratch_types=[
            pltpu.VMEM((batch, ch), jnp.int32),
            pltpu.VMEM((2, batch * GP), jnp.int32),
            pltpu.VMEM((2, GP, latent_dim), jnp.float32),
            pltpu.VMEM((2, batch * GP, latent_dim), jnp.float32),
            [pltpu.SemaphoreType.DMA] * 2,
            [pltpu.SemaphoreType.DMA] * 2,
            [pltpu.SemaphoreType.DMA] * 2,
        ],
    )
    def body(tok_hbm, emb_hbm, pos_hbm, out_hbm, idx_v, gidx, pbuf, bufs,
             gsem, psem, ssem):
        wid = lax.axis_index("s") * NC + lax.axis_index("c")
        s0 = wid * ch
        for b in range(batch):
            pltpu.sync_copy(tok_hbm.at[b * NW + wid], idx_v.at[b])

        def launch(g, k):
            for b in range(batch):
                gidx[k, pl.ds(b * GP, GP)] = idx_v[b, pl.ds(g * GP, GP)]
            gather = pltpu.async_copy(emb_hbm.at[gidx.at[k]], bufs.at[k], gsem[k])
            pstage = pltpu.async_copy(
                pos_hbm.at[pl.ds(s0 + g * GP, GP)], pbuf.at[k], psem[k])
            return gather, pstage

        gathers = [None] * ng
        pstages = [None] * ng
        stores = [None] * ng
        gathers[0], pstages[0] = launch(0, 0)
        for g in range(ng):
            k = g % 2
            buf = bufs.at[k]
            gathers[g].wait()
            if g + 1 < ng:
                if g >= 1:
                    for st in stores[g - 1]:
                        st.wait()
                gathers[g + 1], pstages[g + 1] = launch(g + 1, 1 - k)
            pstages[g].wait()

            def pos_row(p, _, buf=buf, k=k):
                for h in range(2):
                    prow = [
                        pbuf[k, p, pl.ds((h * half + j) * LANES, LANES)]
                        for j in range(half)
                    ]
                    for b in range(batch):
                        r = b * GP + p
                        for j in range(half):
                            sl = pl.ds((h * half + j) * LANES, LANES)
                            buf[r, sl] = buf[r, sl] + prow[j]
                return 0

            lax.fori_loop(0, GP, pos_row, 0)
            stores[g] = [
                pltpu.async_copy(
                    buf.at[pl.ds(b * GP, GP)],
                    out_hbm.at[pl.ds(b * seq_len + s0 + g * GP, GP)],
                    ssem[k],
                )
                for b in range(batch)
            ]
        for g in (ng - 2, ng - 1):
            for st in stores[g]:
                st.wait()

    return body


def kernel(tokens, embed_table, pos_table):
    b, s = tokens.shape
    v, d = embed_table.shape
    tok = tokens.reshape(b * NW, s // NW).astype(jnp.int32)
    out = _make_lookup(b, s, d)(tok, embed_table, pos_table)
    return out.reshape(b, s, d)


# speedup vs baseline: 1.6208x; 1.0354x over previous
import functools

import jax
import jax.numpy as jnp
from jax import lax
from jax.experimental import pallas as pl
from jax.experimental.pallas import tpu as pltpu
from jax.experimental.pallas import tpu_sc as plsc

NC, NS = 2, 16
NW = NC * NS
LANES = 16
GP = 16


def _make_lookup(batch, seq_len, latent_dim):
    ch = seq_len // NW
    ng = ch // GP
    nsl = latent_dim // LANES
    half = nsl // 2
    mesh = plsc.VectorSubcoreMesh(core_axis_name="c", subcore_axis_name="s")

    @functools.partial(
        pl.kernel,
        out_type=jax.ShapeDtypeStruct((batch * seq_len, latent_dim), jnp.float32),
        mesh=mesh,
        scratch_types=[
            pltpu.VMEM((batch, ch), jnp.int32),
            pltpu.VMEM((2, batch * GP), jnp.int32),
            pltpu.VMEM((2, GP, latent_dim), jnp.float32),
            pltpu.VMEM((2, batch * GP, latent_dim), jnp.float32),
            [pltpu.SemaphoreType.DMA] * 2,
            [pltpu.SemaphoreType.DMA] * 2,
            [pltpu.SemaphoreType.DMA] * 2,
        ],
    )
    def body(tok_hbm, emb_hbm, pos_hbm, out_hbm, idx_v, gidx, pbuf, bufs,
             gsem, psem, ssem):
        wid = lax.axis_index("s") * NC + lax.axis_index("c")
        s0 = wid * ch

        def pstage(g, k):
            return pltpu.async_copy(
                pos_hbm.at[pl.ds(s0 + g * GP, GP)], pbuf.at[k], psem[k])

        pstages = [None] * ng
        pstages[0] = pstage(0, 0)
        idx_cps = [
            pltpu.async_copy(tok_hbm.at[b * NW + wid], idx_v.at[b], gsem[1])
            for b in range(batch)
        ]
        for cp in idx_cps:
            cp.wait()

        def launch(g, k):
            for b in range(batch):
                gidx[k, pl.ds(b * GP, GP)] = idx_v[b, pl.ds(g * GP, GP)]
            gather = pltpu.async_copy(emb_hbm.at[gidx.at[k]], bufs.at[k], gsem[k])
            return gather, pstage(g, k) if g > 0 else pstages[0]

        gathers = [None] * ng
        stores = [None] * ng
        gathers[0], pstages[0] = launch(0, 0)
        for g in range(ng):
            k = g % 2
            buf = bufs.at[k]
            gathers[g].wait()
            if g + 1 < ng:
                if g >= 1:
                    for st in stores[g - 1]:
                        st.wait()
                gathers[g + 1], pstages[g + 1] = launch(g + 1, 1 - k)
            pstages[g].wait()

            def pos_row(p, _, buf=buf, k=k):
                for h in range(2):
                    prow = [
                        pbuf[k, p, pl.ds((h * half + j) * LANES, LANES)]
                        for j in range(half)
                    ]
                    for b in range(batch):
                        r = b * GP + p
                        for j in range(half):
                            sl = pl.ds((h * half + j) * LANES, LANES)
                            buf[r, sl] = buf[r, sl] + prow[j]
                return 0

            lax.fori_loop(0, GP, pos_row, 0)
            stores[g] = [
                pltpu.async_copy(
                    buf.at[pl.ds(b * GP, GP)],
                    out_hbm.at[pl.ds(b * seq_len + s0 + g * GP, GP)],
                    ssem[k],
                )
                for b in range(batch)
            ]
        for g in (ng - 2, ng - 1):
            for st in stores[g]:
                st.wait()

    return body


def kernel(tokens, embed_table, pos_table):
    b, s = tokens.shape
    v, d = embed_table.shape
    tok = tokens.reshape(b * NW, s // NW).astype(jnp.int32)
    out = _make_lookup(b, s, d)(tok, embed_table, pos_table)
    return out.reshape(b, s, d)
